# baseline (speedup 1.0000x reference)
import functools

import jax
import jax.numpy as jnp
import numpy as np
from jax import lax
from jax.experimental import pallas as pl
from jax.experimental.pallas import tpu as pltpu
from jax.experimental.pallas import tpu_sc as plsc

_CUTOFF_UPPER = 10.0
_ALPHA = 5.0 / _CUTOFF_UPPER
_RBF_PAD = 64


def _rsqrt_newton(x):
    i = lax.bitcast_convert_type(x, jnp.int32)
    i = jnp.int32(0x5F3759DF) - lax.shift_right_arithmetic(i, 1)
    y = lax.bitcast_convert_type(i, jnp.float32)
    for _ in range(2):
        y = y * (1.5 - 0.5 * x * y * y)
    return y


def _make_sc_gather(n_edges, offset, n_nodes, n_groups):
    info = plsc.get_sparse_core_info()
    nc, ns = info.num_cores, info.num_subcores
    nw = nc * ns
    assert n_edges % (nw * 16) == 0
    ch = n_edges // nw

    mesh = plsc.VectorSubcoreMesh(core_axis_name="c", subcore_axis_name="s")

    @functools.partial(
        pl.kernel,
        mesh=mesh,
        compiler_params=pltpu.CompilerParams(
            needs_layout_passes=False, use_tc_tiling_on_sc=False),
        out_type=(
            jax.ShapeDtypeStruct((n_edges,), jnp.float32),
            jax.ShapeDtypeStruct((n_edges,), jnp.float32),
            jax.ShapeDtypeStruct((n_edges,), jnp.float32),
            jax.ShapeDtypeStruct((n_edges,), jnp.float32),
        ),
        scratch_types=[
            pltpu.VMEM((3 * n_nodes,), jnp.float32),
            pltpu.VMEM((3 * n_groups,), jnp.float32),
            pltpu.VMEM((ch,), jnp.int32),
            pltpu.VMEM((ch,), jnp.int32),
            pltpu.VMEM((ch,), jnp.float32),
            pltpu.VMEM((ch,), jnp.float32),
            pltpu.VMEM((ch,), jnp.float32),
            pltpu.VMEM((ch,), jnp.float32),
        ],
    )
    def sc_gather(ei_hbm, node_hbm, group_hbm,
                  w_out, vx_out, vy_out, vz_out,
                  node_v, group_v, src_v, dst_v, w_v, vx_v, vy_v, vz_v):
        wid = lax.axis_index("s") * nc + lax.axis_index("c")
        base = wid * ch
        pltpu.sync_copy(node_hbm, node_v)
        pltpu.sync_copy(group_hbm, group_v)
        pltpu.sync_copy(ei_hbm.at[0, pl.ds(offset + base, ch)], src_v)
        pltpu.sync_copy(ei_hbm.at[1, pl.ds(offset + base, ch)], dst_v)

        def body(i, carry):
            off = i * 16
            s = src_v[pl.ds(off, 16)]
            d = dst_v[pl.ds(off, 16)]
            nx = plsc.load_gather(node_v, [s])
            ny = plsc.load_gather(node_v, [s + n_nodes])
            nz = plsc.load_gather(node_v, [s + 2 * n_nodes])
            gx = plsc.load_gather(group_v, [d])
            gy = plsc.load_gather(group_v, [d + n_groups])
            gz = plsc.load_gather(group_v, [d + 2 * n_groups])
            dx = nx - gx
            dy = ny - gy
            dz = nz - gz
            w2 = dx * dx + dy * dy + dz * dz
            r = _rsqrt_newton(w2)
            w_v[pl.ds(off, 16)] = w2 * r
            vx_v[pl.ds(off, 16)] = dx * r
            vy_v[pl.ds(off, 16)] = dy * r
            vz_v[pl.ds(off, 16)] = dz * r
            return carry

        lax.fori_loop(0, ch // 16, body, 0, unroll=2)
        pltpu.sync_copy(w_v, w_out.at[pl.ds(base, ch)])
        pltpu.sync_copy(vx_v, vx_out.at[pl.ds(base, ch)])
        pltpu.sync_copy(vy_v, vy_out.at[pl.ds(base, ch)])
        pltpu.sync_copy(vz_v, vz_out.at[pl.ds(base, ch)])

    return sc_gather


def _tc_body(w_ref, vx_ref, vy_ref, vz_ref, means_ref, betas_ref, wmat_ref,
             b_ref, *rest):
    attr_ref, vec_ref = rest[-2], rest[-1]
    block = w_ref.shape[0]
    dist = w_ref[...].reshape(1, block)
    cut = 0.5 * (jnp.cos(dist * np.float32(np.pi / _CUTOFF_UPPER)) + 1.0)
    cut = jnp.where(dist < _CUTOFF_UPPER, cut, 0.0)
    t = jnp.exp(dist * np.float32(-_ALPHA))
    diff = t - means_ref[...]
    rbf_t = cut * jnp.exp(-(betas_ref[...]) * diff * diff)
    attr = lax.dot_general(
        rbf_t.astype(jnp.bfloat16), wmat_ref[...], (((0,), (0,)), ((), ())),
        preferred_element_type=jnp.float32)
    attr_ref[...] = attr + b_ref[...]
    vec_ref[...] = jnp.concatenate(
        [vx_ref[...].reshape(1, block), vy_ref[...].reshape(1, block),
         vz_ref[...].reshape(1, block)], axis=0)


def _tc_rbf_vec(w_flat, vx, vy, vz, means_p, betas_p, wmat_p, b2,
                n_edges, hidden, block, blk_off, prev=None):
    nb = -(-w_flat.shape[0] // block)
    row_spec = pl.BlockSpec((block,), lambda i: (i,))
    const2 = lambda i: (0, 0)
    in_specs = [
        row_spec,
        row_spec,
        row_spec,
        row_spec,
        pl.BlockSpec((_RBF_PAD, 1), const2),
        pl.BlockSpec((_RBF_PAD, 1), const2),
        pl.BlockSpec((_RBF_PAD, hidden), const2),
        pl.BlockSpec((1, hidden), const2),
    ]
    args = [w_flat, vx, vy, vz, means_p, betas_p, wmat_p, b2]
    aliases = {}
    if prev is not None:
        in_specs += [pl.BlockSpec(memory_space=pl.ANY),
                     pl.BlockSpec(memory_space=pl.ANY)]
        args += [prev[0], prev[1]]
        aliases = {8: 0, 9: 1}
    return pl.pallas_call(
        _tc_body,
        grid=(nb,),
        in_specs=in_specs,
        out_specs=(
            pl.BlockSpec((block, hidden), lambda i: (i + blk_off, 0)),
            pl.BlockSpec((3, block), lambda i: (0, i + blk_off)),
        ),
        out_shape=(
            jax.ShapeDtypeStruct((n_edges, hidden), jnp.float32),
            jax.ShapeDtypeStruct((3, n_edges), jnp.float32),
        ),
        input_output_aliases=aliases,
        compiler_params=pltpu.CompilerParams(
            dimension_semantics=("arbitrary",)),
    )(*args)


def kernel(edge_index, node_pos, group_pos, means, betas, W, b):
    n_edges = edge_index.shape[1]
    n_nodes = node_pos.shape[0]
    n_groups = group_pos.shape[0]
    num_rbf, hidden = W.shape

    node_t = node_pos.T.reshape(-1)
    group_t = group_pos.T.reshape(-1)
    ei = edge_index.astype(jnp.int32)

    block = 16384
    half1 = 10 * block
    half2 = n_edges - half1

    sc_a = _make_sc_gather(half1, 0, n_nodes, n_groups)
    sc_b = _make_sc_gather(half2, half1, n_nodes, n_groups)
    w1, vx1, vy1, vz1 = sc_a(ei, node_t, group_t)
    w2, vx2, vy2, vz2 = sc_b(ei, node_t, group_t)

    pad = _RBF_PAD - num_rbf
    means_p = jnp.pad(means, (0, pad)).reshape(_RBF_PAD, 1)
    betas_p = jnp.pad(betas, (0, pad)).reshape(_RBF_PAD, 1)
    wmat_p = jnp.pad(W, ((0, pad), (0, 0))).astype(jnp.bfloat16)
    b2 = b.reshape(1, hidden)

    attr1, vec1 = _tc_rbf_vec(
        w1, vx1, vy1, vz1,
        means_p, betas_p, wmat_p, b2, n_edges, hidden, block, 0)
    edge_attr, vec3 = _tc_rbf_vec(
        w2, vx2, vy2, vz2,
        means_p, betas_p, wmat_p, b2, n_edges, hidden, block, 10,
        prev=(attr1, vec1))
    edge_vec = vec3.T
    edge_weight = jnp.concatenate([w1, w2])

    return (edge_index, edge_weight, edge_attr, edge_vec)


# speedup vs baseline: 1.1885x; 1.1885x over previous
import functools

import jax
import jax.numpy as jnp
import numpy as np
from jax import lax
from jax.experimental import pallas as pl
from jax.experimental.pallas import tpu as pltpu
from jax.experimental.pallas import tpu_sc as plsc

_CUTOFF_UPPER = 10.0
_ALPHA = 5.0 / _CUTOFF_UPPER
_RBF_PAD = 64


def _rsqrt_newton(x):
    i = lax.bitcast_convert_type(x, jnp.int32)
    i = jnp.int32(0x5F3759DF) - lax.shift_right_arithmetic(i, 1)
    y = lax.bitcast_convert_type(i, jnp.float32)
    for _ in range(2):
        y = y * (1.5 - 0.5 * x * y * y)
    return y


def _make_sc_gather(n_edges, n_nodes, n_groups):
    info = plsc.get_sparse_core_info()
    nc, ns = info.num_cores, info.num_subcores
    nw = nc * ns
    assert n_edges % (nw * 16) == 0
    ch = n_edges // nw

    mesh = plsc.VectorSubcoreMesh(core_axis_name="c", subcore_axis_name="s")

    @functools.partial(
        pl.kernel,
        mesh=mesh,
        compiler_params=pltpu.CompilerParams(
            needs_layout_passes=False, use_tc_tiling_on_sc=False),
        out_type=(
            jax.ShapeDtypeStruct((n_edges,), jnp.float32),
            jax.ShapeDtypeStruct((n_edges,), jnp.float32),
            jax.ShapeDtypeStruct((n_edges,), jnp.float32),
            jax.ShapeDtypeStruct((n_edges,), jnp.float32),
        ),
        scratch_types=[
            pltpu.VMEM((3 * n_nodes,), jnp.float32),
            pltpu.VMEM((3 * n_groups,), jnp.float32),
            pltpu.VMEM((ch,), jnp.int32),
            pltpu.VMEM((ch,), jnp.int32),
            pltpu.VMEM((ch,), jnp.float32),
            pltpu.VMEM((ch,), jnp.float32),
            pltpu.VMEM((ch,), jnp.float32),
            pltpu.VMEM((ch,), jnp.float32),
        ],
    )
    def sc_gather(ei_hbm, node_hbm, group_hbm,
                  w_out, vx_out, vy_out, vz_out,
                  node_v, group_v, src_v, dst_v, w_v, vx_v, vy_v, vz_v):
        wid = lax.axis_index("s") * nc + lax.axis_index("c")
        base = wid * ch
        pltpu.sync_copy(node_hbm, node_v)
        pltpu.sync_copy(group_hbm, group_v)
        pltpu.sync_copy(ei_hbm.at[0, pl.ds(base, ch)], src_v)
        pltpu.sync_copy(ei_hbm.at[1, pl.ds(base, ch)], dst_v)

        def body(i, carry):
            off = i * 16
            s = src_v[pl.ds(off, 16)]
            d = dst_v[pl.ds(off, 16)]
            nx = plsc.load_gather(node_v, [s])
            ny = plsc.load_gather(node_v, [s + n_nodes])
            nz = plsc.load_gather(node_v, [s + 2 * n_nodes])
            gx = plsc.load_gather(group_v, [d])
            gy = plsc.load_gather(group_v, [d + n_groups])
            gz = plsc.load_gather(group_v, [d + 2 * n_groups])
            dx = nx - gx
            dy = ny - gy
            dz = nz - gz
            w2 = dx * dx + dy * dy + dz * dz
            r = _rsqrt_newton(w2)
            w_v[pl.ds(off, 16)] = w2 * r
            vx_v[pl.ds(off, 16)] = dx * r
            vy_v[pl.ds(off, 16)] = dy * r
            vz_v[pl.ds(off, 16)] = dz * r
            return carry

        lax.fori_loop(0, ch // 16, body, 0, unroll=2)
        pltpu.sync_copy(w_v, w_out.at[pl.ds(base, ch)])
        pltpu.sync_copy(vx_v, vx_out.at[pl.ds(base, ch)])
        pltpu.sync_copy(vy_v, vy_out.at[pl.ds(base, ch)])
        pltpu.sync_copy(vz_v, vz_out.at[pl.ds(base, ch)])

    return sc_gather


def _tc_body(w_ref, vx_ref, vy_ref, vz_ref, means_ref, betas_ref, wmat_ref,
             b_ref, attr_ref, vec_ref):
    block = w_ref.shape[0]
    dist = w_ref[...].reshape(1, block)
    cut = 0.5 * (jnp.cos(dist * np.float32(np.pi / _CUTOFF_UPPER)) + 1.0)
    cut = jnp.where(dist < _CUTOFF_UPPER, cut, 0.0)
    t = jnp.exp(dist * np.float32(-_ALPHA))
    diff = t - means_ref[...]
    rbf_t = cut * jnp.exp(-(betas_ref[...]) * diff * diff)
    attr = lax.dot_general(
        rbf_t.astype(jnp.bfloat16), wmat_ref[...], (((0,), (0,)), ((), ())),
        preferred_element_type=jnp.float32)
    attr_ref[...] = attr + b_ref[...]
    vec_ref[...] = jnp.concatenate(
        [vx_ref[...].reshape(1, block), vy_ref[...].reshape(1, block),
         vz_ref[...].reshape(1, block)], axis=0)


def _tc_rbf_vec(w_flat, vx, vy, vz, means_p, betas_p, wmat_p, b2,
                n_edges, hidden, block):
    nb = -(-n_edges // block)
    row_spec = pl.BlockSpec((block,), lambda i: (i,))
    const2 = lambda i: (0, 0)
    return pl.pallas_call(
        _tc_body,
        grid=(nb,),
        in_specs=[
            row_spec,
            row_spec,
            row_spec,
            row_spec,
            pl.BlockSpec((_RBF_PAD, 1), const2),
            pl.BlockSpec((_RBF_PAD, 1), const2),
            pl.BlockSpec((_RBF_PAD, hidden), const2),
            pl.BlockSpec((1, hidden), const2),
        ],
        out_specs=(
            pl.BlockSpec((block, hidden), lambda i: (i, 0)),
            pl.BlockSpec((3, block), lambda i: (0, i)),
        ),
        out_shape=(
            jax.ShapeDtypeStruct((n_edges, hidden), jnp.float32),
            jax.ShapeDtypeStruct((3, n_edges), jnp.float32),
        ),
        compiler_params=pltpu.CompilerParams(
            dimension_semantics=("arbitrary",)),
    )(w_flat, vx, vy, vz, means_p, betas_p, wmat_p, b2)


def kernel(edge_index, node_pos, group_pos, means, betas, W, b):
    n_edges = edge_index.shape[1]
    n_nodes = node_pos.shape[0]
    n_groups = group_pos.shape[0]
    num_rbf, hidden = W.shape

    node_t = node_pos.T.reshape(-1)
    group_t = group_pos.T.reshape(-1)

    sc_gather = _make_sc_gather(n_edges, n_nodes, n_groups)
    edge_weight, vx, vy, vz = sc_gather(
        edge_index.astype(jnp.int32), node_t, group_t)

    pad = _RBF_PAD - num_rbf
    means_p = jnp.pad(means, (0, pad)).reshape(_RBF_PAD, 1)
    betas_p = jnp.pad(betas, (0, pad)).reshape(_RBF_PAD, 1)
    wmat_p = jnp.pad(W, ((0, pad), (0, 0))).astype(jnp.bfloat16)
    b2 = b.reshape(1, hidden)

    block = 16384
    edge_attr, vec3 = _tc_rbf_vec(
        edge_weight, vx, vy, vz,
        means_p, betas_p, wmat_p, b2, n_edges, hidden, block)
    edge_vec = vec3.T

    return (edge_index, edge_weight, edge_attr, edge_vec)
